# baseline (speedup 1.0000x reference)
import functools

import jax
import jax.numpy as jnp
from jax import lax
from jax.experimental import pallas as pl
from jax.experimental.pallas import tpu as pltpu
from jax.experimental.pallas import tpu_sc as plsc

_NC = 2
_NS = 16
_NW = _NC * _NS
_CH = 128
_GRP = 2 * _CH
_ALIGN = _NW * _GRP

_H = 128
_N_REG = 100000
_N_IMM = 50000
_N_MEM = 50000
_N_OPS = _N_REG + _N_IMM + _N_MEM


def _wid():
    return lax.axis_index("s") * _NC + lax.axis_index("c")


_DEPTH = 3


def _sc_gather_body(n_pad, table, idx, out, idx_v, rows_v, sems):
    b_per_w = n_pad // _NW
    n_groups = b_per_w // _GRP
    base = _wid() * b_per_w

    def fire(g, par):
        pltpu.sync_copy(idx.at[pl.ds(base + g * _GRP, _GRP)], idx_v.at[par])
        for b in range(2):
            pltpu.async_copy(table.at[idx_v.at[par].at[pl.ds(b * _CH, _CH)]],
                             rows_v.at[par].at[pl.ds(b * _CH, _CH)],
                             sems.at[par])

    fire(0, 0)

    @pl.when(n_groups > 1)
    def _():
        fire(1, 1)

    def step(j, carry):
        par = lax.rem(j, _DEPTH)

        @pl.when(j + _DEPTH - 1 < n_groups)
        def _():
            fire(j + _DEPTH - 1, lax.rem(j + _DEPTH - 1, _DEPTH))

        pltpu.make_async_copy(out.at[pl.ds(0, _GRP)], rows_v.at[par],
                              sems.at[par]).wait()
        pltpu.sync_copy(rows_v.at[par], out.at[pl.ds(base + j * _GRP, _GRP)])
        return carry

    lax.fori_loop(0, n_groups, step, 0, unroll=False)


def _sc_gather(table, idx):
    n_pad = idx.shape[0]
    mesh = plsc.VectorSubcoreMesh(core_axis_name="c", subcore_axis_name="s")
    return pl.kernel(
        functools.partial(_sc_gather_body, n_pad),
        out_type=jax.ShapeDtypeStruct((n_pad, _H), jnp.float32),
        mesh=mesh,
        scratch_types=[
            pltpu.VMEM((_DEPTH, _GRP), jnp.int32),
            pltpu.VMEM((_DEPTH, _GRP, _H), jnp.float32),
            pltpu.SemaphoreType.DMA((_DEPTH,)),
        ],
    )(table, idx)


def _sc_compose_body(n, tab, idx, out, tab_v, idx_v, out_v):
    per_w = n // _NW
    base = _wid() * per_w
    pltpu.sync_copy(tab, tab_v)
    pltpu.sync_copy(idx.at[pl.ds(base, per_w)], idx_v)

    def step(k, carry):
        iv = idx_v[pl.ds(k * 16, 16)]
        out_v[pl.ds(k * 16, 16)] = plsc.load_gather(tab_v, [iv])
        return carry

    lax.fori_loop(0, per_w // 16, step, 0, unroll=False)
    pltpu.sync_copy(out_v, out.at[pl.ds(base, per_w)])


def _sc_compose(tab, idx):
    n = idx.shape[0]
    per_w = n // _NW
    mesh = plsc.VectorSubcoreMesh(core_axis_name="c", subcore_axis_name="s")
    return pl.kernel(
        functools.partial(_sc_compose_body, n),
        out_type=jax.ShapeDtypeStruct((n,), jnp.int32),
        mesh=mesh,
        scratch_types=[
            pltpu.VMEM((tab.shape[0],), jnp.int32),
            pltpu.VMEM((per_w,), jnp.int32),
            pltpu.VMEM((per_w,), jnp.int32),
        ],
        compiler_params=pltpu.CompilerParams(needs_layout_passes=False),
    )(tab, idx)


def _mlp_imm_kernel(x_ref, w1_ref, b1_ref, w2_ref, b2_ref, big_ref, small_ref):
    t = jnp.tanh(x_ref[...])
    h = jnp.maximum(t * w1_ref[...] + b1_ref[...], 0.0)
    y = jnp.dot(h, w2_ref[...], preferred_element_type=jnp.float32) + b2_ref[...]
    big_ref[...] = y
    small_ref[...] = y


def _mlp1_kernel(x_ref, w1_ref, b1_ref, w2_ref, b2_ref, alias_ref, out_ref):
    h = jnp.maximum(
        jnp.dot(x_ref[...], w1_ref[...], preferred_element_type=jnp.float32)
        + b1_ref[...], 0.0)
    out_ref[...] = (
        jnp.dot(h, w2_ref[...], preferred_element_type=jnp.float32)
        + b2_ref[...])


def _mlp4_alias_kernel(x0, x1, x2, x3, w10, w11, w12, w13, b1_ref, w2_ref,
                       b2_ref, alias_ref, out_ref):
    acc = jnp.dot(x0[...], w10[...], preferred_element_type=jnp.float32)
    acc += jnp.dot(x1[...], w11[...], preferred_element_type=jnp.float32)
    acc += jnp.dot(x2[...], w12[...], preferred_element_type=jnp.float32)
    acc += jnp.dot(x3[...], w13[...], preferred_element_type=jnp.float32)
    h = jnp.maximum(acc + b1_ref[...], 0.0)
    out_ref[...] = (
        jnp.dot(h, w2_ref[...], preferred_element_type=jnp.float32)
        + b2_ref[...])


def _mlp5_kernel(x0, x1, x2, x3, x4, w10, w11, w12, w13, w14, b1_ref, w2_ref,
                 b2_ref, out_ref):
    acc = jnp.dot(x0[...], w10[...], preferred_element_type=jnp.float32)
    acc += jnp.dot(x1[...], w11[...], preferred_element_type=jnp.float32)
    acc += jnp.dot(x2[...], w12[...], preferred_element_type=jnp.float32)
    acc += jnp.dot(x3[...], w13[...], preferred_element_type=jnp.float32)
    acc += jnp.dot(x4[...], w14[...], preferred_element_type=jnp.float32)
    h = jnp.maximum(acc + b1_ref[...], 0.0)
    y = (jnp.dot(h, w2_ref[...], preferred_element_type=jnp.float32)
         + b2_ref[...])
    out_ref[...] = y.reshape(out_ref.shape)


def _full(shape):
    return pl.BlockSpec(shape, lambda i: tuple(0 for _ in shape))


def _off_spec(shape, off):
    return pl.BlockSpec(shape, functools.partial(lambda o, i: (i + o, 0), off))


def _spread(n, mod):
    return jnp.arange(n, dtype=jnp.int32) * 37 % jnp.int32(mod)


def kernel(imm, regs, mem_reg0, mem_reg1, mem_imm0, mem_imm1, mnemic,
           mnemic_idx, operand_idx, emb, W_imm1, b_imm1, W_imm2, b_imm2,
           W_reg1, b_reg1, W_reg2, b_reg2, W_mem1, b_mem1, W_mem2, b_mem2,
           W_ins1, b_ins1, W_ins2, b_ins2):
    f32 = jnp.float32
    i32 = jnp.int32
    B, S = mnemic_idx.shape
    n_ins = B * S

    regs = regs.astype(i32)
    mnemic = mnemic.astype(i32)
    mn_idx_flat = mnemic_idx.astype(i32).reshape(-1)
    opi = operand_idx.astype(i32)

    b1_imm = b_imm1.reshape(1, _H)
    b2_imm = b_imm2.reshape(1, _H)
    b1_reg = b_reg1.reshape(1, _H)
    b2_reg = b_reg2.reshape(1, _H)
    b1_mem = b_mem1.reshape(1, _H)
    b2_mem = b_mem2.reshape(1, _H)
    b1_ins = b_ins1.reshape(1, _H)
    b2_ins = b_ins2.reshape(1, _H)

    mn_emb_idx = _sc_compose(mnemic, mn_idx_flat)
    reg_idx = jnp.concatenate([regs, _spread(106496 - _N_REG, emb.shape[0])])
    reg_rows = _sc_gather(emb, reg_idx)

    R = 2000

    buf0, op_imm = pl.pallas_call(
        _mlp_imm_kernel,
        grid=(_N_IMM // R,),
        in_specs=[
            pl.BlockSpec((R, 1), lambda i: (i, 0)),
            _full((1, _H)), _full((1, _H)), _full((_H, _H)), _full((1, _H)),
        ],
        out_specs=[
            _off_spec((R, _H), _N_REG // R),
            _off_spec((R, _H), 0),
        ],
        out_shape=[
            jax.ShapeDtypeStruct((_N_OPS, _H), f32),
            jax.ShapeDtypeStruct((_N_IMM, _H), f32),
        ],
    )(imm, W_imm1, b1_imm, W_imm2, b2_imm)

    _MSL = 52000
    _MPAD = 106496

    def _mem_idx(a0, a1, mod):
        return jnp.concatenate([
            a0.astype(i32), _spread(_MSL - _N_MEM, mod),
            a1.astype(i32), _spread(_MPAD - _MSL - _N_MEM, mod)])

    icat = _sc_gather(op_imm, _mem_idx(mem_imm0, mem_imm1, _N_IMM))
    rcat = _sc_gather(reg_rows, _mem_idx(mem_reg0, mem_reg1, _N_REG))

    mn_rows = _sc_gather(emb, mn_emb_idx)

    buf1 = pl.pallas_call(
        _mlp1_kernel,
        grid=(_N_REG // R,),
        in_specs=[
            _off_spec((R, _H), 0),
            _full((_H, _H)), _full((1, _H)), _full((_H, _H)), _full((1, _H)),
            pl.BlockSpec(memory_space=pl.ANY),
        ],
        out_specs=_off_spec((R, _H), 0),
        out_shape=jax.ShapeDtypeStruct((_N_OPS, _H), f32),
        input_output_aliases={5: 0},
    )(reg_rows, W_reg1, b1_reg, W_reg2, b2_reg, buf0)

    operands = pl.pallas_call(
        _mlp4_alias_kernel,
        grid=(_N_MEM // R,),
        in_specs=[
            _off_spec((R, _H), 0),
            _off_spec((R, _H), _MSL // R),
            _off_spec((R, _H), 0),
            _off_spec((R, _H), _MSL // R),
            _full((_H, _H)), _full((_H, _H)), _full((_H, _H)),
            _full((_H, _H)), _full((1, _H)), _full((_H, _H)),
            _full((1, _H)),
            pl.BlockSpec(memory_space=pl.ANY),
        ],
        out_specs=_off_spec((R, _H), (_N_REG + _N_IMM) // R),
        out_shape=jax.ShapeDtypeStruct((_N_OPS, _H), f32),
        input_output_aliases={11: 0},
    )(rcat, rcat, icat, icat, W_mem1[:_H], W_mem1[_H:2 * _H],
      W_mem1[2 * _H:3 * _H], W_mem1[3 * _H:], b1_mem, W_mem2, b2_mem, buf1)

    idx_q = opi.transpose(2, 0, 1).reshape(-1)
    ops_rows = _sc_gather(operands, idx_q)

    BB = 128
    RB = BB * S
    out = pl.pallas_call(
        _mlp5_kernel,
        grid=(n_ins // RB,),
        in_specs=[
            _off_spec((RB, _H), 0),
            _off_spec((RB, _H), 0),
            _off_spec((RB, _H), n_ins // RB),
            _off_spec((RB, _H), 2 * (n_ins // RB)),
            _off_spec((RB, _H), 3 * (n_ins // RB)),
            _full((_H, _H)), _full((_H, _H)), _full((_H, _H)),
            _full((_H, _H)), _full((_H, _H)),
            _full((1, _H)), _full((_H, _H)), _full((1, _H)),
        ],
        out_specs=pl.BlockSpec((BB, S, _H), lambda i: (i, 0, 0)),
        out_shape=jax.ShapeDtypeStruct((B, S, _H), f32),
    )(mn_rows, ops_rows, ops_rows, ops_rows, ops_rows,
      W_ins1[:_H], W_ins1[_H:2 * _H], W_ins1[2 * _H:3 * _H],
      W_ins1[3 * _H:4 * _H], W_ins1[4 * _H:], b1_ins, W_ins2, b2_ins)

    return out


# speedup vs baseline: 1.0442x; 1.0442x over previous
import functools

import jax
import jax.numpy as jnp
from jax import lax
from jax.experimental import pallas as pl
from jax.experimental.pallas import tpu as pltpu
from jax.experimental.pallas import tpu_sc as plsc

_NC = 2
_NS = 16
_NW = _NC * _NS
_CH = 128
_GRP = 2 * _CH
_ALIGN = _NW * _GRP

_H = 128
_N_REG = 100000
_N_IMM = 50000
_N_MEM = 50000
_N_OPS = _N_REG + _N_IMM + _N_MEM


def _wid():
    return lax.axis_index("s") * _NC + lax.axis_index("c")


_DEPTH = 3


def _sc_gather_body(n_pad, table, idx, out, idx_v, rows_v, sems):
    b_per_w = n_pad // _NW
    n_groups = b_per_w // _GRP
    base = _wid() * b_per_w

    def fire(g, par):
        pltpu.sync_copy(idx.at[pl.ds(base + g * _GRP, _GRP)], idx_v.at[par])
        for b in range(2):
            pltpu.async_copy(table.at[idx_v.at[par].at[pl.ds(b * _CH, _CH)]],
                             rows_v.at[par].at[pl.ds(b * _CH, _CH)],
                             sems.at[par])

    fire(0, 0)

    @pl.when(n_groups > 1)
    def _():
        fire(1, 1)

    def step(j, carry):
        par = lax.rem(j, _DEPTH)

        @pl.when(j + _DEPTH - 1 < n_groups)
        def _():
            fire(j + _DEPTH - 1, lax.rem(j + _DEPTH - 1, _DEPTH))

        pltpu.make_async_copy(out.at[pl.ds(0, _GRP)], rows_v.at[par],
                              sems.at[par]).wait()
        pltpu.sync_copy(rows_v.at[par], out.at[pl.ds(base + j * _GRP, _GRP)])
        return carry

    lax.fori_loop(0, n_groups, step, 0, unroll=False)


def _sc_gather(table, idx):
    n_pad = idx.shape[0]
    mesh = plsc.VectorSubcoreMesh(core_axis_name="c", subcore_axis_name="s")
    return pl.kernel(
        functools.partial(_sc_gather_body, n_pad),
        out_type=jax.ShapeDtypeStruct((n_pad, _H), jnp.float32),
        mesh=mesh,
        scratch_types=[
            pltpu.VMEM((_DEPTH, _GRP), jnp.int32),
            pltpu.VMEM((_DEPTH, _GRP, _H), jnp.float32),
            pltpu.SemaphoreType.DMA((_DEPTH,)),
        ],
    )(table, idx)


def _sc_compose_body(n, tab, idx, out, tab_v, idx_v, out_v):
    per_w = n // _NW
    base = _wid() * per_w
    pltpu.sync_copy(tab, tab_v)
    pltpu.sync_copy(idx.at[pl.ds(base, per_w)], idx_v)

    def step(k, carry):
        iv = idx_v[pl.ds(k * 16, 16)]
        out_v[pl.ds(k * 16, 16)] = plsc.load_gather(tab_v, [iv])
        return carry

    lax.fori_loop(0, per_w // 16, step, 0, unroll=False)
    pltpu.sync_copy(out_v, out.at[pl.ds(base, per_w)])


def _sc_compose(tab, idx):
    n = idx.shape[0]
    per_w = n // _NW
    mesh = plsc.VectorSubcoreMesh(core_axis_name="c", subcore_axis_name="s")
    return pl.kernel(
        functools.partial(_sc_compose_body, n),
        out_type=jax.ShapeDtypeStruct((n,), jnp.int32),
        mesh=mesh,
        scratch_types=[
            pltpu.VMEM((tab.shape[0],), jnp.int32),
            pltpu.VMEM((per_w,), jnp.int32),
            pltpu.VMEM((per_w,), jnp.int32),
        ],
        compiler_params=pltpu.CompilerParams(needs_layout_passes=False),
    )(tab, idx)


def _mlp_imm_kernel(x_ref, w1_ref, b1_ref, w2_ref, b2_ref, big_ref, small_ref):
    t = jnp.tanh(x_ref[pl.ds(pl.program_id(0), 1), :])
    o = lax.dot_general(t, w1_ref[...], (((0,), (0,)), ((), ())),
                        preferred_element_type=jnp.float32)
    h = jnp.maximum(o + b1_ref[...], 0.0)
    y = jnp.dot(h, w2_ref[...], preferred_element_type=jnp.float32) + b2_ref[...]
    big_ref[...] = y
    small_ref[...] = y


def _mlp1_kernel(x_ref, w1_ref, b1_ref, w2_ref, b2_ref, alias_ref, out_ref):
    h = jnp.maximum(
        jnp.dot(x_ref[...], w1_ref[...], preferred_element_type=jnp.float32)
        + b1_ref[...], 0.0)
    out_ref[...] = (
        jnp.dot(h, w2_ref[...], preferred_element_type=jnp.float32)
        + b2_ref[...])


def _mlp4_alias_kernel(x0, x1, x2, x3, w10, w11, w12, w13, b1_ref, w2_ref,
                       b2_ref, alias_ref, out_ref):
    acc = jnp.dot(x0[...], w10[...], preferred_element_type=jnp.float32)
    acc += jnp.dot(x1[...], w11[...], preferred_element_type=jnp.float32)
    acc += jnp.dot(x2[...], w12[...], preferred_element_type=jnp.float32)
    acc += jnp.dot(x3[...], w13[...], preferred_element_type=jnp.float32)
    h = jnp.maximum(acc + b1_ref[...], 0.0)
    out_ref[...] = (
        jnp.dot(h, w2_ref[...], preferred_element_type=jnp.float32)
        + b2_ref[...])


def _mlp5_kernel(x0, x1, x2, x3, x4, w10, w11, w12, w13, w14, b1_ref, w2_ref,
                 b2_ref, out_ref):
    acc = jnp.dot(x0[...], w10[...], preferred_element_type=jnp.float32)
    acc += jnp.dot(x1[...], w11[...], preferred_element_type=jnp.float32)
    acc += jnp.dot(x2[...], w12[...], preferred_element_type=jnp.float32)
    acc += jnp.dot(x3[...], w13[...], preferred_element_type=jnp.float32)
    acc += jnp.dot(x4[...], w14[...], preferred_element_type=jnp.float32)
    h = jnp.maximum(acc + b1_ref[...], 0.0)
    y = (jnp.dot(h, w2_ref[...], preferred_element_type=jnp.float32)
         + b2_ref[...])
    out_ref[...] = y.reshape(out_ref.shape)


def _full(shape):
    return pl.BlockSpec(shape, lambda i: tuple(0 for _ in shape))


def _off_spec(shape, off):
    return pl.BlockSpec(shape, functools.partial(lambda o, i: (i + o, 0), off))


def _spread(n, mod):
    return jnp.arange(n, dtype=jnp.int32) * 37 % jnp.int32(mod)


def kernel(imm, regs, mem_reg0, mem_reg1, mem_imm0, mem_imm1, mnemic,
           mnemic_idx, operand_idx, emb, W_imm1, b_imm1, W_imm2, b_imm2,
           W_reg1, b_reg1, W_reg2, b_reg2, W_mem1, b_mem1, W_mem2, b_mem2,
           W_ins1, b_ins1, W_ins2, b_ins2):
    f32 = jnp.float32
    i32 = jnp.int32
    B, S = mnemic_idx.shape
    n_ins = B * S

    regs = regs.astype(i32)
    mnemic = mnemic.astype(i32)
    mn_idx_flat = mnemic_idx.astype(i32).reshape(-1)
    opi = operand_idx.astype(i32)

    b1_imm = b_imm1.reshape(1, _H)
    b2_imm = b_imm2.reshape(1, _H)
    b1_reg = b_reg1.reshape(1, _H)
    b2_reg = b_reg2.reshape(1, _H)
    b1_mem = b_mem1.reshape(1, _H)
    b2_mem = b_mem2.reshape(1, _H)
    b1_ins = b_ins1.reshape(1, _H)
    b2_ins = b_ins2.reshape(1, _H)

    mn_emb_idx = _sc_compose(mnemic, mn_idx_flat)
    reg_idx = jnp.concatenate([regs, _spread(106496 - _N_REG, emb.shape[0])])
    reg_rows = _sc_gather(emb, reg_idx)

    R = 2000

    imm_row = imm.reshape(_N_IMM // R, R)
    buf0, op_imm = pl.pallas_call(
        _mlp_imm_kernel,
        grid=(_N_IMM // R,),
        in_specs=[
            _full((_N_IMM // R, R)),
            _full((1, _H)), _full((1, _H)), _full((_H, _H)), _full((1, _H)),
        ],
        out_specs=[
            _off_spec((R, _H), _N_REG // R),
            _off_spec((R, _H), 0),
        ],
        out_shape=[
            jax.ShapeDtypeStruct((_N_OPS, _H), f32),
            jax.ShapeDtypeStruct((_N_IMM, _H), f32),
        ],
    )(imm_row, W_imm1, b1_imm, W_imm2, b2_imm)

    _MSL = 52000
    _MPAD = 106496

    def _mem_idx(a0, a1, mod):
        return jnp.concatenate([
            a0.astype(i32), _spread(_MSL - _N_MEM, mod),
            a1.astype(i32), _spread(_MPAD - _MSL - _N_MEM, mod)])

    icat = _sc_gather(op_imm, _mem_idx(mem_imm0, mem_imm1, _N_IMM))
    rcat = _sc_gather(reg_rows, _mem_idx(mem_reg0, mem_reg1, _N_REG))

    mn_rows = _sc_gather(emb, mn_emb_idx)

    buf1 = pl.pallas_call(
        _mlp1_kernel,
        grid=(_N_REG // R,),
        in_specs=[
            _off_spec((R, _H), 0),
            _full((_H, _H)), _full((1, _H)), _full((_H, _H)), _full((1, _H)),
            pl.BlockSpec(memory_space=pl.ANY),
        ],
        out_specs=_off_spec((R, _H), 0),
        out_shape=jax.ShapeDtypeStruct((_N_OPS, _H), f32),
        input_output_aliases={5: 0},
    )(reg_rows, W_reg1, b1_reg, W_reg2, b2_reg, buf0)

    operands = pl.pallas_call(
        _mlp4_alias_kernel,
        grid=(_N_MEM // R,),
        in_specs=[
            _off_spec((R, _H), 0),
            _off_spec((R, _H), _MSL // R),
            _off_spec((R, _H), 0),
            _off_spec((R, _H), _MSL // R),
            _full((_H, _H)), _full((_H, _H)), _full((_H, _H)),
            _full((_H, _H)), _full((1, _H)), _full((_H, _H)),
            _full((1, _H)),
            pl.BlockSpec(memory_space=pl.ANY),
        ],
        out_specs=_off_spec((R, _H), (_N_REG + _N_IMM) // R),
        out_shape=jax.ShapeDtypeStruct((_N_OPS, _H), f32),
        input_output_aliases={11: 0},
    )(rcat, rcat, icat, icat, W_mem1[:_H], W_mem1[_H:2 * _H],
      W_mem1[2 * _H:3 * _H], W_mem1[3 * _H:], b1_mem, W_mem2, b2_mem, buf1)

    idx_q = opi.transpose(2, 0, 1).reshape(-1)
    ops_rows = _sc_gather(operands, idx_q)

    BB = 128
    RB = BB * S
    out = pl.pallas_call(
        _mlp5_kernel,
        grid=(n_ins // RB,),
        in_specs=[
            _off_spec((RB, _H), 0),
            _off_spec((RB, _H), 0),
            _off_spec((RB, _H), n_ins // RB),
            _off_spec((RB, _H), 2 * (n_ins // RB)),
            _off_spec((RB, _H), 3 * (n_ins // RB)),
            _full((_H, _H)), _full((_H, _H)), _full((_H, _H)),
            _full((_H, _H)), _full((_H, _H)),
            _full((1, _H)), _full((_H, _H)), _full((1, _H)),
        ],
        out_specs=pl.BlockSpec((BB, S, _H), lambda i: (i, 0, 0)),
        out_shape=jax.ShapeDtypeStruct((B, S, _H), f32),
    )(mn_rows, ops_rows, ops_rows, ops_rows, ops_rows,
      W_ins1[:_H], W_ins1[_H:2 * _H], W_ins1[2 * _H:3 * _H],
      W_ins1[3 * _H:4 * _H], W_ins1[4 * _H:], b1_ins, W_ins2, b2_ins)

    return out
